# baseline (speedup 1.0000x reference)
import functools

import jax
import jax.numpy as jnp
from jax import lax
from jax.experimental import pallas as pl
from jax.experimental.pallas import tpu as pltpu
from jax.experimental.pallas import tpu_sc as plsc

_N = 16777216
_NC = 2
_NS = 16
_L = 16
_NW = _NC * _NS
_PER_W = _N // _NW
_CHUNK = 16384
_NDMA = _PER_W // _CHUNK


@functools.partial(
    pl.kernel,
    mesh=plsc.VectorSubcoreMesh(core_axis_name="c", subcore_axis_name="s"),
    out_type=jax.ShapeDtypeStruct((_N,), jnp.float32),
    scratch_types=[
        pltpu.VMEM((_CHUNK,), jnp.float32),
        pltpu.VMEM((_L,), jnp.int32),
        pltpu.VMEM((_L,), jnp.float32),
        pltpu.VMEM((_L,), jnp.float32),
        pltpu.SemaphoreType.DMA,
        pltpu.SemaphoreType.DMA,
    ],
)
def _sc_fill(zeros_hbm, idx_hbm, grad_hbm, out_hbm, zbuf, iv, gvec, gtile,
             sem, sem_s):
    c = lax.axis_index("c")
    s = lax.axis_index("s")
    wid = c * _NS + s
    base = wid * _PER_W

    scalar_copies = [
        pltpu.make_async_copy(idx_hbm, iv.at[pl.ds(0, 1)], sem_s),
        pltpu.make_async_copy(grad_hbm, gvec.at[pl.ds(0, 1)], sem_s),
    ]
    for cp in scalar_copies:
        cp.start()

    pltpu.sync_copy(zeros_hbm.at[pl.ds(base, _CHUNK)], zbuf)

    def _fire(j, carry):
        off = pl.multiple_of(base + j * _CHUNK, 8)
        pltpu.make_async_copy(zbuf, out_hbm.at[pl.ds(off, _CHUNK)], sem).start()
        return carry

    lax.fori_loop(0, _NDMA, _fire, 0)

    for cp in scalar_copies:
        cp.wait()
    sidx = iv[...][0] % _N
    g0 = gvec[...][0]

    def _drain(j, carry):
        pltpu.make_async_copy(
            zbuf, out_hbm.at[pl.ds(base, _CHUNK)], sem).wait()
        return carry

    lax.fori_loop(0, _NDMA, _drain, 0)

    @pl.when(sidx // _PER_W == wid)
    def _():
        aligned = jnp.minimum((sidx // 8) * 8, base + _PER_W - _L)
        off = sidx - aligned
        lanes = lax.iota(jnp.int32, _L)
        gtile[...] = jnp.where(lanes == off, g0, 0.0)
        pltpu.sync_copy(gtile, out_hbm.at[pl.ds(aligned, _L)])


def kernel(grad_output, input_sizes, dim, index, out):
    del input_sizes, dim
    idx1 = jnp.asarray(index, jnp.int32).reshape((1,))
    grad1 = jnp.asarray(grad_output, jnp.float32).reshape((1,))
    return _sc_fill(out, idx1, grad1)


# speedup vs baseline: 1.0150x; 1.0150x over previous
import functools

import jax
import jax.numpy as jnp
from jax import lax
from jax.experimental import pallas as pl
from jax.experimental.pallas import tpu as pltpu
from jax.experimental.pallas import tpu_sc as plsc

_N = 16777216
_NC = 2
_NS = 16
_L = 16
_NW = _NC * _NS
_PER_W = _N // _NW
_CHUNK = 16384
_NDMA = _PER_W // _CHUNK


@functools.partial(
    pl.kernel,
    mesh=plsc.VectorSubcoreMesh(core_axis_name="c", subcore_axis_name="s"),
    out_type=jax.ShapeDtypeStruct((_N,), jnp.float32),
    scratch_types=[
        pltpu.VMEM((_CHUNK,), jnp.float32),
        pltpu.VMEM((_L,), jnp.int32),
        pltpu.VMEM((_L,), jnp.float32),
        pltpu.VMEM((_L,), jnp.float32),
        pltpu.SemaphoreType.DMA,
        pltpu.SemaphoreType.DMA,
    ],
)
def _sc_fill(idx_hbm, grad_hbm, out_hbm, zbuf, iv, gvec, gtile, sem, sem_s):
    c = lax.axis_index("c")
    s = lax.axis_index("s")
    wid = c * _NS + s
    base = wid * _PER_W

    scalar_copies = [
        pltpu.make_async_copy(idx_hbm, iv.at[pl.ds(0, 1)], sem_s),
        pltpu.make_async_copy(grad_hbm, gvec.at[pl.ds(0, 1)], sem_s),
    ]
    for cp in scalar_copies:
        cp.start()

    zeros16 = jnp.zeros((_L,), jnp.float32)
    _UNROLL = 16

    def _zero_body(i, carry):
        for j in range(_UNROLL):
            zbuf[pl.ds((i * _UNROLL + j) * _L, _L)] = zeros16
        return carry

    lax.fori_loop(0, _CHUNK // (_L * _UNROLL), _zero_body, 0)

    def _fire(j, carry):
        off = pl.multiple_of(base + j * _CHUNK, 8)
        pltpu.make_async_copy(zbuf, out_hbm.at[pl.ds(off, _CHUNK)], sem).start()
        return carry

    lax.fori_loop(0, _NDMA, _fire, 0)

    for cp in scalar_copies:
        cp.wait()
    sidx = iv[...][0] % _N
    g0 = gvec[...][0]

    def _drain(j, carry):
        pltpu.make_async_copy(
            zbuf, out_hbm.at[pl.ds(base, _CHUNK)], sem).wait()
        return carry

    lax.fori_loop(0, _NDMA, _drain, 0)

    @pl.when(sidx // _PER_W == wid)
    def _():
        aligned = jnp.minimum((sidx // 8) * 8, base + _PER_W - _L)
        off = sidx - aligned
        lanes = lax.iota(jnp.int32, _L)
        gtile[...] = jnp.where(lanes == off, g0, 0.0)
        pltpu.sync_copy(gtile, out_hbm.at[pl.ds(aligned, _L)])


def kernel(grad_output, input_sizes, dim, index, out):
    del input_sizes, dim, out
    idx1 = jnp.asarray(index, jnp.int32).reshape((1,))
    grad1 = jnp.asarray(grad_output, jnp.float32).reshape((1,))
    return _sc_fill(idx1, grad1)


# speedup vs baseline: 1.0522x; 1.0367x over previous
import functools

import jax
import jax.numpy as jnp
from jax import lax
from jax.experimental import pallas as pl
from jax.experimental.pallas import tpu as pltpu
from jax.experimental.pallas import tpu_sc as plsc

_N = 16777216
_L = 16
_CW = 128
_CH = 524288
_NCOPIES = _N // _CH


@functools.partial(
    pl.kernel,
    mesh=plsc.VectorSubcoreMesh(core_axis_name="c", subcore_axis_name="s"),
    out_type=(jax.ShapeDtypeStruct((_CW,), jnp.float32),
              jax.ShapeDtypeStruct((_L,), jnp.int32)),
    scratch_types=[
        pltpu.VMEM((_L,), jnp.int32),
        pltpu.VMEM((_L,), jnp.float32),
        pltpu.VMEM((_CW,), jnp.float32),
        pltpu.VMEM((_L,), jnp.int32),
    ],
)
def _sc_resolve(idx_hbm, grad_hbm, chunk_hbm, meta_hbm, iv, gv, cbuf, mbuf):
    c = lax.axis_index("c")
    s = lax.axis_index("s")

    @pl.when((c == 0) & (s == 0))
    def _():
        pltpu.sync_copy(idx_hbm, iv.at[pl.ds(0, 1)])
        pltpu.sync_copy(grad_hbm, gv.at[pl.ds(0, 1)])
        sidx = iv[...][0] % _N
        g0 = gv[...][0]
        aligned = (sidx // _CW) * _CW
        off = sidx - aligned
        lanes = lax.iota(jnp.int32, _L)
        for j in range(_CW // _L):
            cbuf[pl.ds(j * _L, _L)] = jnp.where(lanes + j * _L == off, g0, 0.0)
        mbuf[...] = jnp.full((_L,), aligned, jnp.int32)
        pltpu.sync_copy(cbuf, chunk_hbm)
        pltpu.sync_copy(mbuf, meta_hbm)


def _fill_body(out_ref, zbuf, sem):
    zbuf[...] = jnp.zeros_like(zbuf)
    copies = [
        pltpu.make_async_copy(zbuf, out_ref.at[pl.ds(k * _CH, _CH)], sem)
        for k in range(_NCOPIES)
    ]
    for cp in copies:
        cp.start()
    for cp in copies:
        cp.wait()


def _commit_body(meta_ref, zeros_ref, chunk_ref, out_ref, sem):
    del zeros_ref
    aligned = pl.multiple_of(meta_ref[0], _CW)
    cp = pltpu.make_async_copy(chunk_ref, out_ref.at[pl.ds(aligned, _CW)], sem)
    cp.start()
    cp.wait()


def kernel(grad_output, input_sizes, dim, index, out):
    del input_sizes, dim, out
    idx1 = jnp.asarray(index, jnp.int32).reshape((1,))
    grad1 = jnp.asarray(grad_output, jnp.float32).reshape((1,))

    chunk, meta = _sc_resolve(idx1, grad1)
    zeros = pl.pallas_call(
        _fill_body,
        out_specs=pl.BlockSpec(memory_space=pl.ANY),
        out_shape=jax.ShapeDtypeStruct((_N,), jnp.float32),
        scratch_shapes=[
            pltpu.VMEM((_CH,), jnp.float32),
            pltpu.SemaphoreType.DMA,
        ],
    )()
    res = pl.pallas_call(
        _commit_body,
        in_specs=[pl.BlockSpec(memory_space=pltpu.SMEM),
                  pl.BlockSpec(memory_space=pl.ANY),
                  pl.BlockSpec(memory_space=pltpu.VMEM)],
        out_specs=pl.BlockSpec(memory_space=pl.ANY),
        out_shape=jax.ShapeDtypeStruct((_N,), jnp.float32),
        input_output_aliases={1: 0},
        scratch_shapes=[pltpu.SemaphoreType.DMA],
    )(meta, zeros, chunk)
    return res


# speedup vs baseline: 1.6257x; 1.5450x over previous
import jax
import jax.numpy as jnp
from jax import lax
from jax.experimental import pallas as pl
from jax.experimental.pallas import tpu as pltpu

_N = 16777216
_CW = 128
_CH = 524288
_NCOPIES = _N // _CH


def _fill_body(out_ref, zbuf, sem):
    zbuf[...] = jnp.zeros_like(zbuf)
    copies = [
        pltpu.make_async_copy(zbuf, out_ref.at[pl.ds(k * _CH, _CH)], sem)
        for k in range(_NCOPIES)
    ]
    for cp in copies:
        cp.start()
    for cp in copies:
        cp.wait()


def _commit_body(idx_ref, grad_ref, zeros_ref, out_ref, cbuf, sem):
    del zeros_ref
    target = idx_ref[0]
    aligned = pl.multiple_of((target // _CW) * _CW, _CW)
    off = target - aligned
    pos = lax.broadcasted_iota(jnp.int32, (_CW,), 0)
    cbuf[...] = jnp.where(pos == off, grad_ref[0], 0.0)
    cp = pltpu.make_async_copy(cbuf, out_ref.at[pl.ds(aligned, _CW)], sem)
    cp.start()
    cp.wait()


def kernel(grad_output, input_sizes, dim, index, out):
    del out
    idx = ((jnp.asarray(index, jnp.int32) + jnp.asarray(dim, jnp.int32))
           % jnp.asarray(input_sizes, jnp.int32)).reshape((1,))
    gval = jnp.asarray(grad_output, jnp.float32).reshape((1,))
    zeros = pl.pallas_call(
        _fill_body,
        out_specs=pl.BlockSpec(memory_space=pl.ANY),
        out_shape=jax.ShapeDtypeStruct((_N,), jnp.float32),
        scratch_shapes=[
            pltpu.VMEM((_CH,), jnp.float32),
            pltpu.SemaphoreType.DMA,
        ],
    )()
    res = pl.pallas_call(
        _commit_body,
        in_specs=[pl.BlockSpec(memory_space=pltpu.SMEM),
                  pl.BlockSpec(memory_space=pltpu.SMEM),
                  pl.BlockSpec(memory_space=pl.ANY)],
        out_specs=pl.BlockSpec(memory_space=pl.ANY),
        out_shape=jax.ShapeDtypeStruct((_N,), jnp.float32),
        input_output_aliases={2: 0},
        scratch_shapes=[
            pltpu.VMEM((_CW,), jnp.float32),
            pltpu.SemaphoreType.DMA,
        ],
    )(idx, gval, zeros)
    return res
